# baseline (device time: 12050 ns/iter reference)
import jax
import jax.numpy as jnp
from jax import lax
from jax.experimental import pallas as pl
from jax.experimental.pallas import tpu as pltpu

M = 512
N_HALF = 512
C = 4
R = M // C


def kernel(x):
    def body(
        x_ref,
        out_ref,
        sv,
        sbuf,
        rbuf,
        kin,
        in_sems,
        keep_sem,
        send_sems,
        recv_sems,
    ):
        my_x = lax.axis_index("x")
        my_y = lax.axis_index("y")
        peer_y = (my_x, 1 - my_y)

        col_keep = my_y * N_HALF
        col_send = (1 - my_y) * N_HALF

        barrier_sem = pltpu.get_barrier_semaphore()
        pl.semaphore_signal(
            barrier_sem,
            inc=1,
            device_id=peer_y,
            device_id_type=pl.DeviceIdType.MESH,
        )

        in_dmas = []
        for c in range(C):
            sl = pl.ds(c * R, R)
            dma = pltpu.make_async_copy(
                x_ref.at[0, sl, pl.ds(col_send, N_HALF)],
                sv.at[sl],
                in_sems.at[c],
            )
            dma.start()
            in_dmas.append(dma)
        keep_dma = pltpu.make_async_copy(
            x_ref.at[0, :, pl.ds(col_keep, N_HALF)], kin, keep_sem
        )
        keep_dma.start()

        rdmas = []
        for c in range(C):
            sl = pl.ds(c * R, R)
            in_dmas[c].wait()
            sbuf[sl] = sv[sl].astype(jnp.bfloat16)
            if c == 0:
                pl.semaphore_wait(barrier_sem, 1)
            rdma = pltpu.make_async_remote_copy(
                src_ref=sbuf.at[sl],
                dst_ref=rbuf.at[sl],
                send_sem=send_sems.at[c],
                recv_sem=recv_sems.at[c],
                device_id=peer_y,
                device_id_type=pl.DeviceIdType.MESH,
            )
            rdma.start()
            rdmas.append(rdma)

        keep_dma.wait()
        for c in range(C):
            sl = pl.ds(c * R, R)
            rdmas[c].wait_recv()
            out_ref[sl, :] = (kin[sl] + rbuf[sl].astype(jnp.float32)).astype(
                jnp.bfloat16
            )

        for c in range(C):
            rdmas[c].wait_send()

    return pl.pallas_call(
        body,
        out_shape=jax.ShapeDtypeStruct((M, N_HALF), jnp.bfloat16),
        in_specs=[pl.BlockSpec(memory_space=pl.ANY)],
        out_specs=pl.BlockSpec(memory_space=pltpu.VMEM),
        scratch_shapes=[
            pltpu.VMEM((M, N_HALF), jnp.float32),
            pltpu.VMEM((M, N_HALF), jnp.bfloat16),
            pltpu.VMEM((M, N_HALF), jnp.bfloat16),
            pltpu.VMEM((M, N_HALF), jnp.float32),
            pltpu.SemaphoreType.DMA((C,)),
            pltpu.SemaphoreType.DMA,
            pltpu.SemaphoreType.DMA((C,)),
            pltpu.SemaphoreType.DMA((C,)),
        ],
        compiler_params=pltpu.CompilerParams(collective_id=0),
    )(x)
